# baseline (speedup 1.0000x reference)
import functools

import jax
import jax.numpy as jnp
from jax import lax
from jax.experimental import pallas as pl
from jax.experimental.pallas import tpu as pltpu
from jax.experimental.pallas import tpu_sc as plsc

B = 1024
K = 200
V = 100000
N = B * K

_NUM_CORES = 2
_NUM_SUBCORES = 16
NW = _NUM_CORES * _NUM_SUBCORES
PER_W = N // NW
LANES = 16
CHUNK = 128
N_CHUNKS = PER_W // CHUNK
STREAMS = 5


@functools.partial(
    pl.kernel,
    out_type=jax.ShapeDtypeStruct((N,), jnp.float32),
    mesh=plsc.VectorSubcoreMesh(core_axis_name="c", subcore_axis_name="s"),
    scratch_types=[
        pltpu.VMEM((PER_W,), jnp.int32),
        pltpu.VMEM((PER_W,), jnp.float32),
        pltpu.SemaphoreType.DMA,
    ],
)
def _sc_gather(scores_hbm, cids_hbm, out_hbm, idx_v, out_v, sem):
    wid = lax.axis_index("s") * _NUM_CORES + lax.axis_index("c")
    base = pl.multiple_of(wid * PER_W, PER_W)
    pltpu.sync_copy(cids_hbm.at[pl.ds(base, PER_W)], idx_v)

    lane = lax.iota(jnp.int32, LANES)

    def to_addr(c):
        for j in range(CHUNK // LANES):
            t = c * (CHUNK // LANES) + j
            sl = pl.ds(t * LANES, LANES)
            p0 = base + t * LANES
            v = idx_v[sl]
            idx_v[sl] = (
                ((v >> 3) << 13)
                + ((v & 7) << 7)
                + ((p0 & 0x1C00) + (p0 & 127) + lane)
            )

    def chunk_copy(c):
        o = pl.multiple_of(c * CHUNK, CHUNK)
        return pltpu.make_async_copy(
            scores_hbm.at[idx_v.at[pl.ds(o, CHUNK)]],
            out_v.at[pl.ds(o, CHUNK)],
            sem,
        )

    QC = N_CHUNKS // STREAMS
    copies = []
    for h in range(STREAMS):
        def q_body(c, carry):
            to_addr(c)
            return carry

        lax.fori_loop(h * QC, (h + 1) * QC, q_body, 0)
        o = pl.multiple_of(h * QC * CHUNK, 8)
        n = QC * CHUNK
        copies.append(pltpu.make_async_copy(
            scores_hbm.at[idx_v.at[pl.ds(o, n)]],
            out_v.at[pl.ds(o, n)],
            sem,
        ))
        copies[-1].start()
    for c in copies:
        c.wait()
    pltpu.sync_copy(out_v, out_hbm.at[pl.ds(base, PER_W)])


def kernel(scores, candidate_ids):
    s_flat = (
        jnp.squeeze(scores, axis=1).T
        .reshape(V // 8, 8, B // 128, 128)
        .transpose(0, 2, 1, 3)
        .reshape(-1)
    )
    c_flat = (
        candidate_ids.T
        .reshape(K // 8, 8, B // 128, 128)
        .transpose(0, 2, 1, 3)
        .reshape(-1)
    )
    out_flat = _sc_gather(s_flat, c_flat)
    return (
        out_flat
        .reshape(K // 8, B // 128, 8, 128)
        .transpose(0, 2, 1, 3)
        .reshape(K, B)
        .T
    )


# speedup vs baseline: 1.0088x; 1.0088x over previous
import functools

import jax
import jax.numpy as jnp
from jax import lax
from jax.experimental import pallas as pl
from jax.experimental.pallas import tpu as pltpu
from jax.experimental.pallas import tpu_sc as plsc

B = 1024
K = 200
V = 100000
N = B * K

_NUM_CORES = 2
_NUM_SUBCORES = 16
NW = _NUM_CORES * _NUM_SUBCORES
PER_W = N // NW
LANES = 16
CHUNK = 128
N_CHUNKS = PER_W // CHUNK
GROUPS = ((0, 2), (2, 4), (6, 8), (14, 12), (26, 12), (38, 12))
SPLIT_G = 3
SPLIT = 14


@functools.partial(
    pl.kernel,
    out_type=jax.ShapeDtypeStruct((N,), jnp.float32),
    mesh=plsc.VectorSubcoreMesh(core_axis_name="c", subcore_axis_name="s"),
    scratch_types=[
        pltpu.VMEM((PER_W,), jnp.int32),
        pltpu.VMEM((PER_W,), jnp.float32),
        pltpu.SemaphoreType.DMA,
        pltpu.SemaphoreType.DMA,
        pltpu.SemaphoreType.DMA,
    ],
)
def _sc_gather(scores_hbm, cids_hbm, out_hbm, idx_v, out_v, sem, in_sem, out_sem):
    wid = lax.axis_index("s") * _NUM_CORES + lax.axis_index("c")
    base = pl.multiple_of(wid * PER_W, PER_W)

    n1 = SPLIT * CHUNK
    in1 = pltpu.make_async_copy(
        cids_hbm.at[pl.ds(base, n1)], idx_v.at[pl.ds(0, n1)], in_sem)
    in2 = pltpu.make_async_copy(
        cids_hbm.at[pl.ds(base + n1, PER_W - n1)],
        idx_v.at[pl.ds(n1, PER_W - n1)], in_sem)
    in1.start()
    in2.start()

    lane = lax.iota(jnp.int32, LANES)

    def to_addr(c):
        for j in range(CHUNK // LANES):
            t = c * (CHUNK // LANES) + j
            sl = pl.ds(t * LANES, LANES)
            p0 = base + t * LANES
            v = idx_v[sl]
            idx_v[sl] = (
                ((v >> 3) << 13)
                + ((v & 7) << 7)
                + ((p0 & 0x1C00) + (p0 & 127) + lane)
            )

    def chunk_copy(c):
        o = pl.multiple_of(c * CHUNK, CHUNK)
        return pltpu.make_async_copy(
            scores_hbm.at[idx_v.at[pl.ds(o, CHUNK)]],
            out_v.at[pl.ds(o, CHUNK)],
            sem,
        )

    def span_copy(c0, nc):
        o = pl.multiple_of(c0 * CHUNK, 8)
        n = nc * CHUNK
        return pltpu.make_async_copy(
            scores_hbm.at[idx_v.at[pl.ds(o, n)]],
            out_v.at[pl.ds(o, n)],
            sem,
        )

    def out_copy(c0, nc):
        o = pl.multiple_of(c0 * CHUNK, 8)
        n = nc * CHUNK
        return pltpu.make_async_copy(
            out_v.at[pl.ds(o, n)],
            out_hbm.at[pl.ds(base + o, n)],
            out_sem,
        )

    def addr_span(c0, nc):
        def q_body(c, carry):
            to_addr(c)
            return carry

        lax.fori_loop(c0, c0 + nc, q_body, 0)

    gathers = []
    in1.wait()
    for c0, nc in GROUPS[:SPLIT_G]:
        addr_span(c0, nc)
        g = span_copy(c0, nc)
        g.start()
        gathers.append((c0, nc, g))
    in2.wait()
    for c0, nc in GROUPS[SPLIT_G:]:
        addr_span(c0, nc)
        g = span_copy(c0, nc)
        g.start()
        gathers.append((c0, nc, g))
    outs = []
    for c0, nc, g in gathers:
        g.wait()
        o = out_copy(c0, nc)
        o.start()
        outs.append(o)
    for o in outs:
        o.wait()


def kernel(scores, candidate_ids):
    s_flat = (
        jnp.squeeze(scores, axis=1).T
        .reshape(V // 8, 8, B // 128, 128)
        .transpose(0, 2, 1, 3)
        .reshape(-1)
    )
    c_flat = (
        candidate_ids.T
        .reshape(K // 8, 8, B // 128, 128)
        .transpose(0, 2, 1, 3)
        .reshape(-1)
    )
    out_flat = _sc_gather(s_flat, c_flat)
    return (
        out_flat
        .reshape(K // 8, B // 128, 8, 128)
        .transpose(0, 2, 1, 3)
        .reshape(K, B)
        .T
    )
